# Initial kernel scaffold; baseline (speedup 1.0000x reference)
import jax
import jax.numpy as jnp
from jax.experimental import pallas as pl


def kernel(x_nchw, w1m, b1m, w2m, b2m, w_fc1_s, b_fc1_s, w_fc2_s, b_fc2_s):
    raise NotImplementedError("write your pallas kernel here")


# speedup vs baseline: 1.1879x; 1.1879x over previous
import functools

import jax
import jax.numpy as jnp
from jax.experimental import pallas as pl
from jax.experimental.pallas import tpu as pltpu

_VMEM_LIMIT = 64 * 1024 * 1024


def _conv_pool_kernel(p_ref, w_ref, b_ref, o_ref):
    w = w_ref[...]
    a = jnp.maximum(
        jnp.dot(p_ref[0], w, preferred_element_type=jnp.float32),
        jnp.dot(p_ref[1], w, preferred_element_type=jnp.float32))
    b = jnp.maximum(
        jnp.dot(p_ref[2], w, preferred_element_type=jnp.float32),
        jnp.dot(p_ref[3], w, preferred_element_type=jnp.float32))
    o_ref[...] = jnp.maximum(jnp.maximum(a, b) + b_ref[...], 0.0)


def _fc_fused_kernel(x_ref, w1_ref, b1_ref, w2_ref, o_ref, acc_ref):
    k = pl.program_id(1)

    @pl.when(k == 0)
    def _():
        acc_ref[...] = jnp.zeros_like(acc_ref)

    acc_ref[...] += jnp.dot(x_ref[...], w1_ref[0],
                            preferred_element_type=jnp.float32)

    @pl.when(k == pl.num_programs(1) - 1)
    def _():
        h = jnp.maximum(acc_ref[...] + b1_ref[0], 0.0)
        o_ref[0] = jnp.dot(h, w2_ref[0], preferred_element_type=jnp.float32)


def _conv3x3_relu_pool(x_nhwc, w_mat, b_row, tm):
    N, H, W, C = x_nhwc.shape
    K = 9 * C
    Cout = w_mat.shape[1]

    xp = jnp.pad(x_nhwc, ((0, 0), (1, 1), (1, 1), (0, 0)))
    groups = []
    for dh in (0, 1):
        for dw in (0, 1):
            cols = [xp[:, dh + kh: dh + kh + H: 2, dw + kw: dw + kw + W: 2, :]
                    for kh in range(3) for kw in range(3)]
            groups.append(jnp.concatenate(cols, axis=-1)
                          .reshape(N * (H // 2) * (W // 2), K))
    patches = jnp.stack(groups, axis=0)
    Np = patches.shape[1]

    out = pl.pallas_call(
        _conv_pool_kernel,
        out_shape=jax.ShapeDtypeStruct((Np, Cout), jnp.float32),
        grid=(Np // tm,),
        in_specs=[pl.BlockSpec((4, tm, K), lambda i: (0, i, 0)),
                  pl.BlockSpec((K, Cout), lambda i: (0, 0)),
                  pl.BlockSpec((1, Cout), lambda i: (0, 0))],
        out_specs=pl.BlockSpec((tm, Cout), lambda i: (i, 0)),
        compiler_params=pltpu.CompilerParams(
            dimension_semantics=("parallel",),
            vmem_limit_bytes=_VMEM_LIMIT),
    )(patches, w_mat.astype(jnp.bfloat16), b_row)
    return out.reshape(N, H // 2, W // 2, Cout)


def _fc_fused(x, w1_s, b1_s, w2_rows, b2, tk):
    M, K = x.shape
    S, _, BN = w1_s.shape
    nk = K // tk

    parts = pl.pallas_call(
        _fc_fused_kernel,
        out_shape=jax.ShapeDtypeStruct((S, M, 5), jnp.float32),
        grid=(S, nk),
        in_specs=[pl.BlockSpec((M, tk), lambda s, k: (0, k)),
                  pl.BlockSpec((1, tk, BN), lambda s, k: (s, k, 0)),
                  pl.BlockSpec((1, 1, BN), lambda s, k: (s, 0, 0)),
                  pl.BlockSpec((1, BN, 5), lambda s, k: (s, 0, 0))],
        out_specs=pl.BlockSpec((1, M, 5), lambda s, k: (s, 0, 0)),
        scratch_shapes=[pltpu.VMEM((M, BN), jnp.float32)],
        compiler_params=pltpu.CompilerParams(
            dimension_semantics=("parallel", "arbitrary"),
            vmem_limit_bytes=_VMEM_LIMIT),
    )(x, w1_s, b1_s, w2_rows)
    return parts.sum(axis=0) + b2


def kernel(x_nchw, w1m, b1m, w2m, b2m, w_fc1_s, b_fc1_s, w_fc2_s, b_fc2_s):
    N = x_nchw.shape[0]
    x = jnp.transpose(x_nchw, (0, 2, 3, 1)).astype(jnp.bfloat16)

    z1 = _conv3x3_relu_pool(x, w1m, b1m, tm=2048)
    z2 = _conv3x3_relu_pool(z1.astype(jnp.bfloat16), w2m, b2m, tm=1024)

    flat = jnp.transpose(z2, (0, 3, 1, 2)).reshape(N, -1)

    S, _, BN = w_fc1_s.shape
    w2_rows = w_fc2_s[0].reshape(S, BN, -1)
    return _fc_fused(flat, w_fc1_s, b_fc1_s, w2_rows, b_fc2_s[0, 0],
                     tk=14336)
